# baseline (speedup 1.0000x reference)
import functools

import jax
import jax.numpy as jnp
from jax import lax
from jax.experimental import pallas as pl
from jax.experimental.pallas import tpu as pltpu
from jax.experimental.pallas import tpu_sc as plsc

_info = plsc.get_sparse_core_info()
_NC, _NS = _info.num_cores, _info.num_subcores
_NW = _NC * _NS
_CH = 128


def _make_gather(B, N, DQ, DT):
    b_per_w = B // _NW
    n_ch = b_per_w // _CH
    mesh = plsc.VectorSubcoreMesh(core_axis_name="c", subcore_axis_name="s")

    @functools.partial(
        pl.kernel,
        mesh=mesh,
        compiler_params=pltpu.CompilerParams(
            use_tc_tiling_on_sc=False, needs_layout_passes=False
        ),
        out_type=(
            jax.ShapeDtypeStruct((DQ, B), jnp.float32),
            jax.ShapeDtypeStruct((DT, B), jnp.float32),
        ),
        scratch_types=[
            pltpu.VMEM((n_ch, _CH), jnp.int32),
            pltpu.VMEM((DQ, b_per_w), jnp.float32),
            pltpu.VMEM((DT, b_per_w), jnp.float32),
            pltpu.SemaphoreType.DMA,
            pltpu.SemaphoreType.DMA,
        ],
    )
    def gather(q_hbm, t_hbm, idx_hbm, q_out, t_out, idx_v, qg_v, tg_v, sem_q, sem_t):
        wid = lax.axis_index("s") * _NC + lax.axis_index("c")
        base = wid * b_per_w
        pltpu.sync_copy(idx_hbm.at[pl.ds(wid * n_ch, n_ch)], idx_v)
        copies = []
        for c in range(DQ):
            col = q_hbm.at[c]
            for j in range(n_ch):
                copies.append(
                    pltpu.async_copy(
                        col.at[idx_v.at[j]],
                        qg_v.at[c].at[pl.ds(j * _CH, _CH)],
                        sem_q,
                    )
                )
        for c in range(DT):
            col = t_hbm.at[c]
            for j in range(n_ch):
                copies.append(
                    pltpu.async_copy(
                        col.at[idx_v.at[j]],
                        tg_v.at[c].at[pl.ds(j * _CH, _CH)],
                        sem_t,
                    )
                )
        for cp in copies:
            cp.wait()
        for c in range(DQ):
            pltpu.sync_copy(qg_v.at[c], q_out.at[c].at[pl.ds(base, b_per_w)])
        for c in range(DT):
            pltpu.sync_copy(tg_v.at[c], t_out.at[c].at[pl.ds(base, b_per_w)])

    return gather


def kernel(q_pointcloud_camera_table, t_pointcloud_camera_table, camera_pose_indices):
    B = camera_pose_indices.shape[0]
    N, DQ = q_pointcloud_camera_table.shape
    DT = t_pointcloud_camera_table.shape[1]
    idx = camera_pose_indices.astype(jnp.int32)
    q_out, t_out = _make_gather(B, N, DQ, DT)(
        q_pointcloud_camera_table.T,
        t_pointcloud_camera_table.T,
        idx.reshape(B // _CH, _CH),
    )
    return q_out.T, t_out.T
